# baseline (speedup 1.0000x reference)
import functools

import jax
import jax.numpy as jnp
from jax import lax
from jax.experimental import pallas as pl
from jax.experimental.pallas import tpu as pltpu
from jax.experimental.pallas import tpu_sc as plsc

VOCAB = 100000
H = 768
B = 1024
S = 50
N = B * S
NW = 32
PER_W = N // NW
CH = 32
BCH = B // CH
NCH = PER_W // CH
DEPTH = 2
EPS = 1e-5
TCN = 1024
SROW = 1


def _sc_gather(table_hbm, idx_hbm, out_hbm, idx_v, bufs, sems):
    wid = lax.axis_index("s") * 2 + lax.axis_index("c")
    base = wid * NCH
    pltpu.sync_copy(idx_hbm.at[pl.ds(base * CH, NCH * CH)], idx_v)

    def g_start(t, b):
        pltpu.async_copy(table_hbm.at[idx_v.at[pl.ds(t * CH, CH)]],
                         bufs[b], sems[b])

    def g_wait(t, b):
        pltpu.make_async_copy(table_hbm.at[idx_v.at[pl.ds(t * CH, CH)]],
                              bufs[b], sems[b]).wait()

    def g_out(t, b):
        gt = base + t
        s = gt // BCH
        b0 = (gt % BCH) * CH
        pltpu.sync_copy(bufs[b], out_hbm.at[s, pl.ds(b0, CH)])

    for b in range(DEPTH):
        g_start(b, b)

    def outer(g, _):
        for b in range(DEPTH):
            t = g * DEPTH + b
            g_wait(t, b)
            g_out(t, b)
            nxt = t + DEPTH

            @pl.when(nxt < NCH)
            def _():
                g_start(nxt, b)
        return _
    lax.fori_loop(0, NCH // DEPTH, outer, None)


def _gather_rows(fixed_ans_emb, idx_pm):
    mesh = plsc.VectorSubcoreMesh(core_axis_name="c", subcore_axis_name="s")
    fn = pl.kernel(
        _sc_gather,
        mesh=mesh,
        out_type=jax.ShapeDtypeStruct((S, B, H), jnp.float32),
        scratch_types=[
            pltpu.VMEM((PER_W,), jnp.int32),
            [pltpu.VMEM((CH, H), jnp.float32) for _ in range(DEPTH)],
            [pltpu.SemaphoreType.DMA for _ in range(DEPTH)],
        ],
    )
    return fn(fixed_ans_emb, idx_pm)


def _tc_ln(gath_ref, pos_ref, gpos_ref, bpos_ref, gans_ref, bans_ref,
           out_ref, padd_ref):
    si = pl.program_id(0)

    @pl.when((si == 0) & (pl.program_id(1) == 0))
    def _():
        p = pos_ref[...]
        m = jnp.mean(p, axis=1, keepdims=True)
        d = p - m
        v = jnp.mean(d * d, axis=1, keepdims=True)
        padd_ref[...] = (d * lax.rsqrt(v + EPS) * gpos_ref[...]
                         + bpos_ref[...] + bans_ref[...])

    x = gath_ref[...]
    m = jnp.mean(x, axis=2, keepdims=True)
    v = jnp.mean(x * x, axis=2, keepdims=True) - m * m
    out_ref[...] = ((x - m) * lax.rsqrt(v + EPS) * gans_ref[...][None]
                    + padd_ref[pl.ds(si * SROW, SROW)][:, None, :])


def _ln_posadd(gathered, pos_table, gpos2, bpos2, gans2, bans2):
    return pl.pallas_call(
        _tc_ln,
        grid=(S // SROW, B // TCN),
        in_specs=[
            pl.BlockSpec((SROW, TCN, H), lambda i, j: (i, j, 0)),
            pl.BlockSpec((S, H), lambda i, j: (0, 0)),
            pl.BlockSpec((1, H), lambda i, j: (0, 0)),
            pl.BlockSpec((1, H), lambda i, j: (0, 0)),
            pl.BlockSpec((1, H), lambda i, j: (0, 0)),
            pl.BlockSpec((1, H), lambda i, j: (0, 0)),
        ],
        out_specs=pl.BlockSpec((SROW, TCN, H), lambda i, j: (i, j, 0)),
        out_shape=jax.ShapeDtypeStruct((S, B, H), jnp.float32),
        scratch_shapes=[pltpu.VMEM((S, H), jnp.float32)],
    )(gathered, pos_table, gpos2, bpos2, gans2, bans2)


@jax.jit
def _prev_embedding(fixed_ans_emb, idx_pm, pos_table, ln_pos_g, ln_pos_b,
                    ln_ans_g, ln_ans_b):
    gathered = _gather_rows(fixed_ans_emb, idx_pm)
    z = _ln_posadd(gathered, pos_table, ln_pos_g.reshape(1, H),
                   ln_pos_b.reshape(1, H), ln_ans_g.reshape(1, H),
                   ln_ans_b.reshape(1, H))
    return jnp.transpose(z, (1, 0, 2))


def kernel(fixed_ans_emb, prev_inds, pos_table, ln_pos_g, ln_pos_b,
           ln_ans_g, ln_ans_b):
    idx_pm = prev_inds.astype(jnp.int32).T.reshape(-1)
    return _prev_embedding(fixed_ans_emb, idx_pm, pos_table, ln_pos_g,
                           ln_pos_b, ln_ans_g, ln_ans_b)


# speedup vs baseline: 1.0053x; 1.0053x over previous
import jax
import jax.numpy as jnp
from jax import lax
from jax.experimental import pallas as pl
from jax.experimental.pallas import tpu as pltpu
from jax.experimental.pallas import tpu_sc as plsc

VOCAB = 100000
H = 768
B = 1024
S = 50
SHALF = S // 2
NW = 32
CH = 32
BCH = B // CH
NCH = SHALF * BCH // NW
DEPTH = 2
EPS = 1e-5


def _sc_gather(table_hbm, idx_hbm, out_hbm, idx_v, bufs, sems):
    wid = lax.axis_index("s") * 2 + lax.axis_index("c")
    base = wid * NCH
    pltpu.sync_copy(idx_hbm.at[pl.ds(base * CH, NCH * CH)], idx_v)

    def g_start(t, b):
        pltpu.async_copy(table_hbm.at[idx_v.at[pl.ds(t * CH, CH)]],
                         bufs[b], sems[b])

    def g_wait(t, b):
        pltpu.make_async_copy(table_hbm.at[idx_v.at[pl.ds(t * CH, CH)]],
                              bufs[b], sems[b]).wait()

    def g_out(t, b):
        gt = base + t
        s = gt // BCH
        b0 = (gt % BCH) * CH
        pltpu.sync_copy(bufs[b], out_hbm.at[s, pl.ds(b0, CH)])

    for b in range(DEPTH):
        g_start(b, b)

    def outer(g, _):
        for b in range(DEPTH):
            t = g * DEPTH + b
            g_wait(t, b)
            g_out(t, b)
            nxt = t + DEPTH

            @pl.when(nxt < NCH)
            def _():
                g_start(nxt, b)
        return _
    lax.fori_loop(0, NCH // DEPTH, outer, None)
    t_last = NCH - 1
    g_wait(t_last, t_last % DEPTH)
    g_out(t_last, t_last % DEPTH)


def _gather_rows(fixed_ans_emb, idx_half):
    mesh = plsc.VectorSubcoreMesh(core_axis_name="c", subcore_axis_name="s")
    fn = pl.kernel(
        _sc_gather,
        mesh=mesh,
        out_type=jax.ShapeDtypeStruct((SHALF, B, H), jnp.float32),
        scratch_types=[
            pltpu.VMEM((NCH * CH,), jnp.int32),
            [pltpu.VMEM((CH, H), jnp.float32) for _ in range(DEPTH)],
            [pltpu.SemaphoreType.DMA for _ in range(DEPTH)],
        ],
    )
    return fn(fixed_ans_emb, idx_half)


def _make_tc_ln(qoff):
    def _tc_ln(gath_ref, pos_ref, gpos_ref, bpos_ref, gans_ref, bans_ref,
               *rest):
        out_ref, padd_ref = rest[-2], rest[-1]
        si = pl.program_id(0)

        @pl.when(si == 0)
        def _():
            p = pos_ref[...]
            m = jnp.mean(p, axis=1, keepdims=True)
            d = p - m
            v = jnp.mean(d * d, axis=1, keepdims=True)
            padd_ref[...] = (d * lax.rsqrt(v + EPS) * gpos_ref[...]
                             + bpos_ref[...] + bans_ref[...])

        x = gath_ref[...]
        m = jnp.mean(x, axis=2, keepdims=True)
        v = jnp.mean(x * x, axis=2, keepdims=True) - m * m
        out_ref[...] = ((x - m) * lax.rsqrt(v + EPS) * gans_ref[...][None]
                        + padd_ref[pl.ds(si + qoff, 1)][None])
    return _tc_ln


_SMALL_SPECS = [
    pl.BlockSpec((S, H), lambda i: (0, 0)),
    pl.BlockSpec((1, H), lambda i: (0, 0)),
    pl.BlockSpec((1, H), lambda i: (0, 0)),
    pl.BlockSpec((1, H), lambda i: (0, 0)),
    pl.BlockSpec((1, H), lambda i: (0, 0)),
]


def _ln_half0(g0, pos_table, gpos2, bpos2, gans2, bans2):
    return pl.pallas_call(
        _make_tc_ln(0),
        grid=(SHALF,),
        in_specs=[pl.BlockSpec((1, B, H), lambda i: (i, 0, 0))]
        + _SMALL_SPECS,
        out_specs=pl.BlockSpec((1, B, H), lambda i: (i, 0, 0)),
        out_shape=jax.ShapeDtypeStruct((S, B, H), jnp.float32),
        scratch_shapes=[pltpu.VMEM((S, H), jnp.float32)],
    )(g0, pos_table, gpos2, bpos2, gans2, bans2)


def _ln_half1(g1, big, pos_table, gpos2, bpos2, gans2, bans2):
    return pl.pallas_call(
        _make_tc_ln(SHALF),
        grid=(SHALF,),
        in_specs=[pl.BlockSpec((1, B, H), lambda i: (i, 0, 0))]
        + _SMALL_SPECS
        + [pl.BlockSpec(memory_space=pl.ANY)],
        out_specs=pl.BlockSpec((1, B, H), lambda i: (i + SHALF, 0, 0)),
        out_shape=jax.ShapeDtypeStruct((S, B, H), jnp.float32),
        scratch_shapes=[pltpu.VMEM((S, H), jnp.float32)],
        input_output_aliases={6: 0},
    )(g1, pos_table, gpos2, bpos2, gans2, bans2, big)


@jax.jit
def _prev_embedding(fixed_ans_emb, idx_pm, pos_table, ln_pos_g, ln_pos_b,
                    ln_ans_g, ln_ans_b):
    gpos2 = ln_pos_g.reshape(1, H)
    bpos2 = ln_pos_b.reshape(1, H)
    gans2 = ln_ans_g.reshape(1, H)
    bans2 = ln_ans_b.reshape(1, H)
    nh = SHALF * B
    g0 = _gather_rows(fixed_ans_emb, idx_pm[:nh])
    g1 = _gather_rows(fixed_ans_emb, idx_pm[nh:])
    big = _ln_half0(g0, pos_table, gpos2, bpos2, gans2, bans2)
    z = _ln_half1(g1, big, pos_table, gpos2, bpos2, gans2, bans2)
    return jnp.transpose(z, (1, 0, 2))


def kernel(fixed_ans_emb, prev_inds, pos_table, ln_pos_g, ln_pos_b,
           ln_ans_g, ln_ans_b):
    idx_pm = prev_inds.astype(jnp.int32).T.reshape(-1)
    return _prev_embedding(fixed_ans_emb, idx_pm, pos_table, ln_pos_g,
                           ln_pos_b, ln_ans_g, ln_ans_b)
